# baseline (speedup 1.0000x reference)
import jax
import jax.numpy as jnp
from jax.experimental import pallas as pl
from jax.experimental.pallas import tpu as pltpu

_PACK = 8
_BLOCK_ROWS = 8192


def _packed_mlp_kernel(x_ref, w1p_ref, b1p_ref, w2p_ref, b2p_ref, o_ref,
                       xs_ref, ys_ref):
    tb, d_in = x_ref.shape
    d_out = o_ref.shape[1]
    g = tb // _PACK
    xs_ref[...] = x_ref[...].reshape(g, _PACK, d_in)
    xp = xs_ref[...].reshape(g, _PACK * d_in)
    h = jnp.dot(xp, w1p_ref[...], preferred_element_type=jnp.float32)
    h = jnp.maximum(h + b1p_ref[...], 0.0)
    y = jnp.dot(h, w2p_ref[...], preferred_element_type=jnp.float32)
    y = y + b2p_ref[...]
    ys_ref[...] = y.reshape(g, _PACK, d_out)
    o_ref[...] = ys_ref[...].reshape(tb, d_out).astype(o_ref.dtype)


def kernel(x, w1, b1, w2, b2):
    B, d_in = x.shape
    d_hidden = w1.shape[1]
    d_out = w2.shape[1]
    P = _PACK

    eye = jnp.eye(P, dtype=jnp.float32)
    w1p = jnp.kron(eye, w1.astype(jnp.float32))
    b1p = jnp.tile(b1.astype(jnp.float32), (1, P))
    w2p = jnp.kron(eye, w2.astype(jnp.float32))
    b2p = jnp.tile(b2.astype(jnp.float32), (1, P))

    tb = _BLOCK_ROWS
    while B % tb != 0:
        tb //= 2
    grid = (B // tb,)

    vmem = pltpu.MemorySpace.VMEM
    out = pl.pallas_call(
        _packed_mlp_kernel,
        out_shape=jax.ShapeDtypeStruct((B, d_out), x.dtype),
        grid=grid,
        in_specs=[
            pl.BlockSpec((tb, d_in), lambda i: (i, 0), memory_space=vmem),
            pl.BlockSpec((P * d_in, P * d_hidden), lambda i: (0, 0), memory_space=vmem),
            pl.BlockSpec((1, P * d_hidden), lambda i: (0, 0), memory_space=vmem),
            pl.BlockSpec((P * d_hidden, P * d_out), lambda i: (0, 0), memory_space=vmem),
            pl.BlockSpec((1, P * d_out), lambda i: (0, 0), memory_space=vmem),
        ],
        out_specs=pl.BlockSpec((tb, d_out), lambda i: (i, 0), memory_space=vmem),
        scratch_shapes=[
            pltpu.VMEM((tb // P, P, d_in), jnp.float32),
            pltpu.VMEM((tb // P, P, d_out), jnp.float32),
        ],
        compiler_params=pltpu.CompilerParams(
            dimension_semantics=("parallel",),
        ),
    )(x, w1p, b1p, w2p, b2p)

    return out


# speedup vs baseline: 1.0807x; 1.0807x over previous
import jax
import jax.numpy as jnp
from jax.experimental import pallas as pl
from jax.experimental.pallas import tpu as pltpu

_PACK = 8
_BLOCK_GROUPS = 8192


def _packed_mlp_kernel(x_ref, w1p_ref, b1p_ref, w2p_ref, b2p_ref, o_ref):
    h = jnp.dot(x_ref[...], w1p_ref[...], preferred_element_type=jnp.float32)
    h = jnp.maximum(h + b1p_ref[...], 0.0)
    y = jnp.dot(h, w2p_ref[...], preferred_element_type=jnp.float32)
    o_ref[...] = (y + b2p_ref[...]).astype(o_ref.dtype)


def kernel(x, w1, b1, w2, b2):
    B, d_in = x.shape
    d_hidden = w1.shape[1]
    d_out = w2.shape[1]
    P = _PACK

    eye = jnp.eye(P, dtype=jnp.float32)
    w1p = jnp.kron(eye, w1.astype(jnp.float32))
    b1p = jnp.tile(b1.astype(jnp.float32), (1, P))
    w2p = jnp.kron(eye, w2.astype(jnp.float32))
    b2p = jnp.tile(b2.astype(jnp.float32), (1, P))

    G = B // P
    xp = x.reshape(G, P * d_in)

    gb = _BLOCK_GROUPS
    while G % gb != 0:
        gb //= 2
    grid = (G // gb,)

    vmem = pltpu.MemorySpace.VMEM
    outp = pl.pallas_call(
        _packed_mlp_kernel,
        out_shape=jax.ShapeDtypeStruct((G, P * d_out), x.dtype),
        grid=grid,
        in_specs=[
            pl.BlockSpec((gb, P * d_in), lambda i: (i, 0), memory_space=vmem),
            pl.BlockSpec((P * d_in, P * d_hidden), lambda i: (0, 0), memory_space=vmem),
            pl.BlockSpec((1, P * d_hidden), lambda i: (0, 0), memory_space=vmem),
            pl.BlockSpec((P * d_hidden, P * d_out), lambda i: (0, 0), memory_space=vmem),
            pl.BlockSpec((1, P * d_out), lambda i: (0, 0), memory_space=vmem),
        ],
        out_specs=pl.BlockSpec((gb, P * d_out), lambda i: (i, 0), memory_space=vmem),
        compiler_params=pltpu.CompilerParams(
            dimension_semantics=("parallel",),
        ),
    )(xp, w1p, b1p, w2p, b2p)

    return outp.reshape(B, d_out)
